# baseline (speedup 1.0000x reference)
import functools

import jax
import jax.numpy as jnp
from jax import lax
from jax.experimental import pallas as pl
from jax.experimental.pallas import tpu as pltpu
from jax.experimental.pallas import tpu_sc as plsc

FLOAT_MAX = 1e8
N_PTS = 20000
N_PAD = 20480
N_BOXES = 512
CAP = 128
KEEP = 48
LANES = 16
NVREG = N_PAD // LANES
NGRP = 4
VPG = NVREG // NGRP
LCAP = 32

_MESH = plsc.VectorSubcoreMesh(core_axis_name="c", subcore_axis_name="s")
_PARAMS = pltpu.CompilerParams(needs_layout_passes=False)


def _sorted16(x):
    return lax.sort(x)


def _rev(x):
    return lax.rev(x, (0,))


def _merge2(a, b):
    br = _rev(b)
    lo = jnp.minimum(a, br)
    hi = jnp.maximum(a, br)
    return _sorted16(lo), _sorted16(hi)


def _merge4(a0, a1, b0, b1):
    l0 = jnp.minimum(a0, _rev(b1))
    l1 = jnp.minimum(a1, _rev(b0))
    h0 = jnp.maximum(a0, _rev(b1))
    h1 = jnp.maximum(a1, _rev(b0))
    s0 = _sorted16(jnp.minimum(l0, l1))
    s1 = _sorted16(jnp.maximum(l0, l1))
    s2 = _sorted16(jnp.minimum(h0, h1))
    s3 = _sorted16(jnp.maximum(h0, h1))
    return s0, s1, s2, s3


def _select33v(v):
    s = [_sorted16(x) for x in v]
    a = _merge2(s[0], s[1])
    b = _merge2(s[2], s[3])
    c = _merge2(s[4], s[5])
    d = _merge2(s[6], s[7])
    e = _merge4(*a, *b)
    f = _merge4(*c, *d)
    l0 = jnp.minimum(e[0], _rev(f[3]))
    l1 = jnp.minimum(e[1], _rev(f[2]))
    l2 = jnp.minimum(e[2], _rev(f[1]))
    l3 = jnp.minimum(e[3], _rev(f[0]))
    x0 = jnp.minimum(l0, l2)
    x1 = jnp.minimum(l1, l3)
    y0 = jnp.maximum(l0, l2)
    y1 = jnp.maximum(l1, l3)
    c0 = _sorted16(jnp.minimum(x0, x1))
    c1 = _sorted16(jnp.maximum(x0, x1))
    thr = jnp.minimum(jnp.min(y0), jnp.min(y1))
    return c0, c1, thr


def _squash33(v6, inf16):
    return _select33v(v6 + [inf16, inf16])


NB = 8
NSW = 16 // NB


@functools.partial(
    pl.kernel,
    out_type=jax.ShapeDtypeStruct((N_BOXES,), jnp.float32),
    mesh=_MESH,
    compiler_params=_PARAMS,
    scratch_types=[
        pltpu.VMEM((N_PAD,), jnp.float32),
        pltpu.VMEM((N_PAD,), jnp.float32),
        pltpu.VMEM((N_PAD,), jnp.float32),
        pltpu.VMEM((N_PAD,), jnp.int32),
        pltpu.VMEM((N_BOXES + 16,), jnp.float32),
        pltpu.VMEM((N_BOXES + 16,), jnp.float32),
        pltpu.VMEM((N_BOXES + 16,), jnp.float32),
        pltpu.VMEM((N_BOXES + 16,), jnp.int32),
        pltpu.VMEM((16 * NGRP * LANES,), jnp.float32),
        pltpu.VMEM((16 * LANES * LCAP,), jnp.float32),
        pltpu.VMEM((16 * LANES,), jnp.int32),
        pltpu.VMEM((2 * LANES,), jnp.float32),
        pltpu.VMEM((CAP,), jnp.float32),
        pltpu.VMEM((LANES,), jnp.float32),
    ],
)
def _k1_thresholds(px_h, py_h, pz_h, lev_h, cx_h, cy_h, cz_h, bl_h,
                   thr_h, px_v, py_v, pz_v, lev_v, cx_v, cy_v, cz_v, bl_v,
                   p_v, can_v, cnt_v, u_sc, buf_v, stage_v):
    wid = lax.axis_index("s") * 2 + lax.axis_index("c")
    pltpu.sync_copy(px_h, px_v.at[pl.ds(0, N_PAD)])
    pltpu.sync_copy(py_h, py_v.at[pl.ds(0, N_PAD)])
    pltpu.sync_copy(pz_h, pz_v.at[pl.ds(0, N_PAD)])
    pltpu.sync_copy(lev_h, lev_v.at[pl.ds(0, N_PAD)])
    pltpu.sync_copy(cx_h, cx_v.at[pl.ds(0, N_BOXES)])
    pltpu.sync_copy(cy_h, cy_v.at[pl.ds(0, N_BOXES)])
    pltpu.sync_copy(cz_h, cz_v.at[pl.ds(0, N_BOXES)])
    pltpu.sync_copy(bl_h, bl_v.at[pl.ds(0, N_BOXES)])

    lanes = lax.iota(jnp.int32, LANES)
    lane_base = lanes * LCAP
    inf16 = jnp.full((LANES,), jnp.inf, jnp.float32)

    cxw = cx_v[pl.ds(wid * 16, LANES)]
    cyw = cy_v[pl.ds(wid * 16, LANES)]
    czw = cz_v[pl.ds(wid * 16, LANES)]
    blw = bl_v[pl.ds(wid * 16, LANES)]

    def lev_cnt(j, c01):
        c0, c1 = c01
        levv = lev_v[pl.ds(j * LANES, LANES)]
        return (c0 + (levv == 0).astype(jnp.int32),
                c1 + (levv == 1).astype(jnp.int32))

    c0v, c1v = lax.fori_loop(0, NVREG, lev_cnt,
                             (jnp.zeros((LANES,), jnp.int32),) * 2, unroll=4)
    n0 = jnp.sum(c0v)
    n01 = n0 + jnp.sum(c1v)

    def _lev_range(bl_t):
        lo = jnp.where(bl_t == 0, 0, jnp.where(bl_t == 1, n0 // LANES, 0))
        hi = jnp.where(bl_t == 0, (n0 + LANES - 1) // LANES,
                       jnp.where(bl_t == 1, (n01 + LANES - 1) // LANES, 0))
        return lo, hi

    sweep_lo = []
    sweep_m = []
    for s in range(NSW):
        bxs = [cxw[s * NB + t] for t in range(NB)]
        bys = [cyw[s * NB + t] for t in range(NB)]
        bzs = [czw[s * NB + t] for t in range(NB)]
        bls = [blw[s * NB + t] for t in range(NB)]
        lo_s = jnp.int32(NVREG)
        hi_s = jnp.int32(0)
        for t in range(NB):
            lo_t, hi_t = _lev_range(bls[t])
            lo_s = jnp.minimum(lo_s, lo_t)
            hi_s = jnp.maximum(hi_s, hi_t)
        hi_s = jnp.maximum(hi_s, lo_s)
        vpg_s = (hi_s - lo_s + NGRP - 1) // NGRP
        sweep_lo.append((lo_s, hi_s, vpg_s))
        sweep_m.append(N_PAD - LANES * (hi_s - lo_s))

        def grp_body(g, _):
            def p1(j, mns):
                base = (lo_s + g * vpg_s + j) * LANES
                pxv = px_v[pl.ds(base, LANES)]
                pyv = py_v[pl.ds(base, LANES)]
                pzv = pz_v[pl.ds(base, LANES)]
                levv = lev_v[pl.ds(base, LANES)]
                new = []
                for t in range(NB):
                    dx = pxv - bxs[t]
                    dy = pyv - bys[t]
                    dz = pzv - bzs[t]
                    d = (dx * dx + dy * dy) + dz * dz
                    cd = jnp.where(levv == bls[t], d, FLOAT_MAX)
                    new.append(jnp.minimum(mns[t], cd))
                return tuple(new)

            nb_g = jnp.clip(hi_s - lo_s - g * vpg_s, 0, vpg_s)
            mns = lax.fori_loop(0, nb_g, p1, (inf16,) * NB)
            for t in range(NB):
                p_v[pl.ds(((s * NB + t) * NGRP) * LANES + g * LANES,
                          LANES)] = mns[t]
            return _

        lax.fori_loop(0, NGRP, grp_body, 0)

    m0, m1 = sweep_m[0], sweep_m[-1]
    m_v = jnp.where(lanes < NB, m0, m1)

    def ub_body(bb, uacc):
        v4 = [p_v[pl.ds((bb * NGRP + k) * LANES, LANES)]
              for k in range(NGRP)]
        a0, a1 = _merge2(_sorted16(v4[0]), _sorted16(v4[1]))
        b0, b1 = _merge2(_sorted16(v4[2]), _sorted16(v4[3]))
        h0 = jnp.maximum(a0, _rev(b1))
        h1 = jnp.maximum(a1, _rev(b0))
        ub = jnp.minimum(jnp.min(h0), jnp.min(h1))
        return jnp.where(lanes == bb, ub, uacc)

    uacc = lax.fori_loop(0, 16, ub_body, inf16)
    uacc = jnp.where(m_v >= 33, jnp.minimum(uacc, FLOAT_MAX), uacc)
    u_sc[pl.ds(0, LANES)] = uacc
    u_sc[pl.ds(LANES, LANES)] = inf16

    def init_can(k, _):
        can_v[pl.ds(k * LANES, LANES)] = inf16
        return _

    lax.fori_loop(0, 16 * LCAP, init_can, 0, unroll=8)
    for s in range(NSW):
        bxs = [cxw[s * NB + t] for t in range(NB)]
        bys = [cyw[s * NB + t] for t in range(NB)]
        bzs = [czw[s * NB + t] for t in range(NB)]
        bls = [blw[s * NB + t] for t in range(NB)]
        ubs = [uacc[s * NB + t] for t in range(NB)]
        lo_s, hi_s, _vpg = sweep_lo[s]

        def p2(j, cnts):
            base = j * LANES
            pxv = px_v[pl.ds(base, LANES)]
            pyv = py_v[pl.ds(base, LANES)]
            pzv = pz_v[pl.ds(base, LANES)]
            levv = lev_v[pl.ds(base, LANES)]
            new = []
            for t in range(NB):
                dx = pxv - bxs[t]
                dy = pyv - bys[t]
                dz = pzv - bzs[t]
                d = (dx * dx + dy * dy) + dz * dz
                cd = jnp.where(levv == bls[t], d, FLOAT_MAX)
                m = cd <= ubs[t]
                idx = (lane_base + jnp.minimum(cnts[t], LCAP - 1)
                       + (s * NB + t) * LANES * LCAP)
                plsc.store_scatter(can_v, [idx], cd, mask=m)
                new.append(cnts[t] + m.astype(jnp.int32))
            return tuple(new)

        cnts = lax.fori_loop(lo_s, hi_s, p2,
                             (jnp.zeros((LANES,), jnp.int32),) * NB)
        for t in range(NB):
            cnt_v[pl.ds((s * NB + t) * LANES, LANES)] = cnts[t]

    def box_body(bb, thrv):
        mx = jnp.max(cnt_v[pl.ds(bb * LANES, LANES)])
        u_b = u_sc[pl.ds(bb, LANES)][0]
        m_b = jnp.where(bb < NB, m0, m1)
        sent_ok = ((m_b == 0) | (u_b < FLOAT_MAX) | (m_b >= 33))

        def fast(_):
            r = []
            for q in range(4):
                cv = [can_v[pl.ds((bb * LCAP + q * 8 + k) * LANES, LANES)]
                      for k in range(8)]
                c0, c1, t = _select33v(cv)
                r += [c0, c1, jnp.full((LANES,), t, jnp.float32)]
            c0, c1, t = _squash33(r[0:6], inf16)
            d0, d1, t2 = _squash33(r[6:12], inf16)
            _, _, thr_b = _squash33(
                [c0, c1, jnp.full((LANES,), t, jnp.float32),
                 d0, d1, jnp.full((LANES,), t2, jnp.float32)], inf16)
            return thr_b

        def slow(_):
            gb = wid * 16 + bb
            bx = cx_v[pl.ds(gb, LANES)][0]
            by = cy_v[pl.ds(gb, LANES)][0]
            bz = cz_v[pl.ds(gb, LANES)][0]
            bl = bl_v[pl.ds(gb, LANES)][0]
            for j in range(8):
                buf_v[pl.ds(j * LANES, LANES)] = inf16

            def flush(thr, count):
                c0, c1, t = _select33v(
                    [buf_v[pl.ds(j * LANES, LANES)] for j in range(8)])
                buf_v[pl.ds(0, LANES)] = c0
                buf_v[pl.ds(LANES, LANES)] = c1
                buf_v[pl.ds(2 * LANES, LANES)] = jnp.full(
                    (LANES,), t, jnp.float32)
                for j in range(3, 8):
                    buf_v[pl.ds(j * LANES, LANES)] = inf16
                return t, jnp.int32(KEEP)

            def sbody(i, carry):
                thr, count = carry
                base = i * LANES
                dx = px_v[pl.ds(base, LANES)] - bx
                dy = py_v[pl.ds(base, LANES)] - by
                dz = pz_v[pl.ds(base, LANES)] - bz
                d = (dx * dx + dy * dy) + dz * dz
                cd = jnp.where(lev_v[pl.ds(base, LANES)] == bl, d, FLOAT_MAX)
                mask = cd < thr
                cnt2 = jnp.sum(mask.astype(jnp.int32))
                thr, count = lax.cond(count > CAP - LANES, flush,
                                      lambda t, c: (t, c), thr, count)
                plsc.store_compressed(buf_v.at[pl.ds(count, LANES)], cd,
                                      mask=mask)
                return thr, count + cnt2

            lax.fori_loop(0, NVREG, sbody,
                          (jnp.float32(jnp.inf), jnp.int32(0)))
            _, _, thr_b = _select33v(
                [buf_v[pl.ds(j * LANES, LANES)] for j in range(8)])
            return thr_b

        use_slow = (mx > LCAP) | jnp.logical_not(sent_ok)
        thr_b = lax.cond(use_slow, slow, fast, 0)
        adj = jnp.logical_not(use_slow) & (m_b > 0) & (u_b >= FLOAT_MAX)
        thr_b = jnp.where(adj, jnp.minimum(thr_b, FLOAT_MAX), thr_b)
        return jnp.where(lanes == bb, thr_b, thrv)

    thrv = lax.fori_loop(0, 16, box_body, inf16)
    stage_v[...] = thrv
    pltpu.sync_copy(stage_v, thr_h.at[pl.ds(wid * 16, 16)])


@functools.partial(
    pl.kernel,
    out_type=jax.ShapeDtypeStruct((N_PAD,), jnp.int32),
    mesh=_MESH,
    compiler_params=_PARAMS,
    scratch_types=[
        pltpu.VMEM((640,), jnp.float32),
        pltpu.VMEM((640,), jnp.float32),
        pltpu.VMEM((640,), jnp.float32),
        pltpu.VMEM((640,), jnp.int32),
        pltpu.VMEM((N_BOXES,), jnp.float32),
        pltpu.VMEM((N_BOXES,), jnp.float32),
        pltpu.VMEM((N_BOXES,), jnp.float32),
        pltpu.VMEM((N_BOXES,), jnp.int32),
        pltpu.VMEM((N_BOXES,), jnp.float32),
        pltpu.VMEM((640,), jnp.int32),
    ],
)
def _k2_assign(px_h, py_h, pz_h, lev_h, cx_h, cy_h, cz_h, bl_h, thr_in_h,
               out_h, px_v, py_v, pz_v, lev_v, cx_v, cy_v, cz_v, bl_v,
               thr_v, out_v):
    wid = lax.axis_index("s") * 2 + lax.axis_index("c")
    base = wid * 640
    pltpu.sync_copy(px_h.at[pl.ds(base, 640)], px_v)
    pltpu.sync_copy(py_h.at[pl.ds(base, 640)], py_v)
    pltpu.sync_copy(pz_h.at[pl.ds(base, 640)], pz_v)
    pltpu.sync_copy(lev_h.at[pl.ds(base, 640)], lev_v)
    pltpu.sync_copy(cx_h, cx_v)
    pltpu.sync_copy(cy_h, cy_v)
    pltpu.sync_copy(cz_h, cz_v)
    pltpu.sync_copy(bl_h, bl_v)
    pltpu.sync_copy(thr_in_h, thr_v)

    inf16 = jnp.full((LANES,), jnp.inf, jnp.float32)
    zero16 = jnp.zeros((LANES,), jnp.int32)
    VT = 4

    def vec_body(v, _):
        offs = [(v * VT + i) * LANES for i in range(VT)]
        pxs = [px_v[pl.ds(o, LANES)] for o in offs]
        pys = [py_v[pl.ds(o, LANES)] for o in offs]
        pzs = [pz_v[pl.ds(o, LANES)] for o in offs]
        levs = [lev_v[pl.ds(o, LANES)] for o in offs]

        def group_body(g, carry):
            mn, idx, mn2, idx2 = carry
            cxg = cx_v[pl.ds(g * LANES, LANES)]
            cyg = cy_v[pl.ds(g * LANES, LANES)]
            czg = cz_v[pl.ds(g * LANES, LANES)]
            blg = bl_v[pl.ds(g * LANES, LANES)]
            thg = thr_v[pl.ds(g * LANES, LANES)]
            mn, idx, mn2, idx2 = list(mn), list(idx), list(mn2), list(idx2)
            for j in range(LANES):
                b = g * LANES + j
                bx = cxg[j]
                by = cyg[j]
                bz = czg[j]
                bl = blg[j]
                th = thg[j]
                for i in range(VT):
                    dx = pxs[i] - bx
                    dy = pys[i] - by
                    dz = pzs[i] - bz
                    d = (dx * dx + dy * dy) + dz * dz
                    lt1 = d < mn[i]
                    mn[i] = jnp.where(lt1, d, mn[i])
                    idx[i] = jnp.where(lt1, b, idx[i])
                    cd = jnp.where(levs[i] == bl, d, FLOAT_MAX)
                    cd2 = jnp.where(cd < th, d, FLOAT_MAX)
                    lt2 = cd2 < mn2[i]
                    mn2[i] = jnp.where(lt2, cd2, mn2[i])
                    idx2[i] = jnp.where(lt2, b, idx2[i])
            return tuple(mn), tuple(idx), tuple(mn2), tuple(idx2)

        mn, idx, mn2, idx2 = lax.fori_loop(
            0, N_BOXES // LANES, group_body,
            ((inf16,) * VT, (zero16,) * VT, (inf16,) * VT, (zero16,) * VT))
        for i in range(VT):
            ok = (mn2[i] < FLOAT_MAX) & (idx2[i] == idx[i])
            out_v[pl.ds(offs[i], LANES)] = jnp.where(ok, idx2[i],
                                                     jnp.int32(-1))
        return _

    lax.fori_loop(0, 640 // (LANES * VT), vec_body, 0)
    pltpu.sync_copy(out_v, out_h.at[pl.ds(base, 640)])


def kernel(points, gt_boxes, levels, gt_labels):
    pad = N_PAD - N_PTS
    px = jnp.pad(points[:, 0], (0, pad))
    py = jnp.pad(points[:, 1], (0, pad))
    pz = jnp.pad(points[:, 2], (0, pad))
    lev = jnp.pad(levels, (0, pad), constant_values=-1)
    cx = gt_boxes[:, 0]
    cy = gt_boxes[:, 1]
    cz = gt_boxes[:, 2]
    bl = jnp.asarray([0], jnp.int32)[gt_labels]
    thr = _k1_thresholds(px, py, pz, lev, cx, cy, cz, bl)
    out = _k2_assign(px, py, pz, lev, cx, cy, cz, bl, thr)
    return out[:N_PTS]


# speedup vs baseline: 1.0161x; 1.0161x over previous
import functools

import jax
import jax.numpy as jnp
from jax import lax
from jax.experimental import pallas as pl
from jax.experimental.pallas import tpu as pltpu
from jax.experimental.pallas import tpu_sc as plsc

FLOAT_MAX = 1e8
N_PTS = 20000
N_PAD = 20480
N_BOXES = 512
CAP = 128
KEEP = 48
LANES = 16
NVREG = N_PAD // LANES
NGRP = 4
VPG = NVREG // NGRP
LCAP = 32

_MESH = plsc.VectorSubcoreMesh(core_axis_name="c", subcore_axis_name="s")
_PARAMS = pltpu.CompilerParams(needs_layout_passes=False)


def _sorted16(x):
    return lax.sort(x)


def _rev(x):
    return lax.rev(x, (0,))


def _merge2(a, b):
    br = _rev(b)
    lo = jnp.minimum(a, br)
    hi = jnp.maximum(a, br)
    return _sorted16(lo), _sorted16(hi)


def _merge4(a0, a1, b0, b1):
    l0 = jnp.minimum(a0, _rev(b1))
    l1 = jnp.minimum(a1, _rev(b0))
    h0 = jnp.maximum(a0, _rev(b1))
    h1 = jnp.maximum(a1, _rev(b0))
    s0 = _sorted16(jnp.minimum(l0, l1))
    s1 = _sorted16(jnp.maximum(l0, l1))
    s2 = _sorted16(jnp.minimum(h0, h1))
    s3 = _sorted16(jnp.maximum(h0, h1))
    return s0, s1, s2, s3


def _select33v(v):
    s = [_sorted16(x) for x in v]
    a = _merge2(s[0], s[1])
    b = _merge2(s[2], s[3])
    c = _merge2(s[4], s[5])
    d = _merge2(s[6], s[7])
    e = _merge4(*a, *b)
    f = _merge4(*c, *d)
    l0 = jnp.minimum(e[0], _rev(f[3]))
    l1 = jnp.minimum(e[1], _rev(f[2]))
    l2 = jnp.minimum(e[2], _rev(f[1]))
    l3 = jnp.minimum(e[3], _rev(f[0]))
    x0 = jnp.minimum(l0, l2)
    x1 = jnp.minimum(l1, l3)
    y0 = jnp.maximum(l0, l2)
    y1 = jnp.maximum(l1, l3)
    c0 = _sorted16(jnp.minimum(x0, x1))
    c1 = _sorted16(jnp.maximum(x0, x1))
    thr = jnp.minimum(jnp.min(y0), jnp.min(y1))
    return c0, c1, thr


def _squash33(v6, inf16):
    return _select33v(v6 + [inf16, inf16])


NB = 8
NSW = 16 // NB


@functools.partial(
    pl.kernel,
    out_type=jax.ShapeDtypeStruct((N_BOXES,), jnp.float32),
    mesh=_MESH,
    compiler_params=_PARAMS,
    scratch_types=[
        pltpu.VMEM((N_PAD,), jnp.float32),
        pltpu.VMEM((N_PAD,), jnp.float32),
        pltpu.VMEM((N_PAD,), jnp.float32),
        pltpu.VMEM((N_PAD,), jnp.int32),
        pltpu.VMEM((N_BOXES + 16,), jnp.float32),
        pltpu.VMEM((N_BOXES + 16,), jnp.float32),
        pltpu.VMEM((N_BOXES + 16,), jnp.float32),
        pltpu.VMEM((N_BOXES + 16,), jnp.int32),
        pltpu.VMEM((16 * NGRP * LANES,), jnp.float32),
        pltpu.VMEM((16 * LANES * LCAP,), jnp.float32),
        pltpu.VMEM((16 * LANES,), jnp.int32),
        pltpu.VMEM((2 * LANES,), jnp.float32),
        pltpu.VMEM((CAP,), jnp.float32),
        pltpu.VMEM((LANES,), jnp.float32),
    ],
)
def _k1_thresholds(px_h, py_h, pz_h, lev_h, cx_h, cy_h, cz_h, bl_h,
                   thr_h, px_v, py_v, pz_v, lev_v, cx_v, cy_v, cz_v, bl_v,
                   p_v, can_v, cnt_v, u_sc, buf_v, stage_v):
    wid = lax.axis_index("s") * 2 + lax.axis_index("c")
    pltpu.sync_copy(px_h, px_v.at[pl.ds(0, N_PAD)])
    pltpu.sync_copy(py_h, py_v.at[pl.ds(0, N_PAD)])
    pltpu.sync_copy(pz_h, pz_v.at[pl.ds(0, N_PAD)])
    pltpu.sync_copy(lev_h, lev_v.at[pl.ds(0, N_PAD)])
    pltpu.sync_copy(cx_h, cx_v.at[pl.ds(0, N_BOXES)])
    pltpu.sync_copy(cy_h, cy_v.at[pl.ds(0, N_BOXES)])
    pltpu.sync_copy(cz_h, cz_v.at[pl.ds(0, N_BOXES)])
    pltpu.sync_copy(bl_h, bl_v.at[pl.ds(0, N_BOXES)])

    lanes = lax.iota(jnp.int32, LANES)
    lane_base = lanes * LCAP
    inf16 = jnp.full((LANES,), jnp.inf, jnp.float32)

    cxw = cx_v[pl.ds(wid * 16, LANES)]
    cyw = cy_v[pl.ds(wid * 16, LANES)]
    czw = cz_v[pl.ds(wid * 16, LANES)]
    blw = bl_v[pl.ds(wid * 16, LANES)]

    def lev_cnt(j, c01):
        c0, c1 = c01
        levv = lev_v[pl.ds(j * LANES, LANES)]
        return (c0 + (levv == 0).astype(jnp.int32),
                c1 + (levv == 1).astype(jnp.int32))

    c0v, c1v = lax.fori_loop(0, NVREG, lev_cnt,
                             (jnp.zeros((LANES,), jnp.int32),) * 2, unroll=4)
    n0 = jnp.sum(c0v)
    n01 = n0 + jnp.sum(c1v)

    def _lev_range(bl_t):
        lo = jnp.where(bl_t == 0, 0, jnp.where(bl_t == 1, n0 // LANES, 0))
        hi = jnp.where(bl_t == 0, (n0 + LANES - 1) // LANES,
                       jnp.where(bl_t == 1, (n01 + LANES - 1) // LANES, 0))
        return lo, hi

    sweep_lo = []
    sweep_m = []
    for s in range(NSW):
        bxs = [cxw[s * NB + t] for t in range(NB)]
        bys = [cyw[s * NB + t] for t in range(NB)]
        bzs = [czw[s * NB + t] for t in range(NB)]
        bls = [blw[s * NB + t] for t in range(NB)]
        lo_s = jnp.int32(NVREG)
        hi_s = jnp.int32(0)
        for t in range(NB):
            lo_t, hi_t = _lev_range(bls[t])
            lo_s = jnp.minimum(lo_s, lo_t)
            hi_s = jnp.maximum(hi_s, hi_t)
        hi_s = jnp.maximum(hi_s, lo_s)
        vpg_s = (hi_s - lo_s + NGRP - 1) // NGRP
        sweep_lo.append((lo_s, hi_s, vpg_s))
        sweep_m.append(N_PAD - LANES * (hi_s - lo_s))

        def grp_body(g, _):
            def p1(j, mns):
                base = (lo_s + g * vpg_s + j) * LANES
                pxv = px_v[pl.ds(base, LANES)]
                pyv = py_v[pl.ds(base, LANES)]
                pzv = pz_v[pl.ds(base, LANES)]
                levv = lev_v[pl.ds(base, LANES)]
                new = []
                for t in range(NB):
                    dx = pxv - bxs[t]
                    dy = pyv - bys[t]
                    dz = pzv - bzs[t]
                    d = (dx * dx + dy * dy) + dz * dz
                    cd = jnp.where(levv == bls[t], d, FLOAT_MAX)
                    new.append(jnp.minimum(mns[t], cd))
                return tuple(new)

            nb_g = jnp.clip(hi_s - lo_s - g * vpg_s, 0, vpg_s)
            mns = lax.fori_loop(0, nb_g, p1, (inf16,) * NB)
            for t in range(NB):
                p_v[pl.ds(((s * NB + t) * NGRP) * LANES + g * LANES,
                          LANES)] = mns[t]
            return _

        lax.fori_loop(0, NGRP, grp_body, 0)

    m0, m1 = sweep_m[0], sweep_m[-1]
    m_v = jnp.where(lanes < NB, m0, m1)

    def ub_body(bb, uacc):
        pva = [p_v[pl.ds((bb * NGRP + k) * LANES, LANES)]
               for k in range(NGRP)] + [inf16] * (8 - NGRP)
        _, _, ub = _select33v(pva)
        return jnp.where(lanes == bb, ub, uacc)

    uacc = lax.fori_loop(0, 16, ub_body, inf16)
    uacc = jnp.where(m_v >= 33, jnp.minimum(uacc, FLOAT_MAX), uacc)
    u_sc[pl.ds(0, LANES)] = uacc
    u_sc[pl.ds(LANES, LANES)] = inf16

    def init_can(k, _):
        can_v[pl.ds(k * LANES, LANES)] = inf16
        return _

    lax.fori_loop(0, 16 * LCAP, init_can, 0, unroll=8)
    for s in range(NSW):
        bxs = [cxw[s * NB + t] for t in range(NB)]
        bys = [cyw[s * NB + t] for t in range(NB)]
        bzs = [czw[s * NB + t] for t in range(NB)]
        bls = [blw[s * NB + t] for t in range(NB)]
        ubs = [uacc[s * NB + t] for t in range(NB)]
        lo_s, hi_s, _vpg = sweep_lo[s]

        def p2(j, cnts):
            base = j * LANES
            pxv = px_v[pl.ds(base, LANES)]
            pyv = py_v[pl.ds(base, LANES)]
            pzv = pz_v[pl.ds(base, LANES)]
            levv = lev_v[pl.ds(base, LANES)]
            new = []
            for t in range(NB):
                dx = pxv - bxs[t]
                dy = pyv - bys[t]
                dz = pzv - bzs[t]
                d = (dx * dx + dy * dy) + dz * dz
                cd = jnp.where(levv == bls[t], d, FLOAT_MAX)
                m = cd <= ubs[t]
                idx = (lane_base + jnp.minimum(cnts[t], LCAP - 1)
                       + (s * NB + t) * LANES * LCAP)
                plsc.store_scatter(can_v, [idx], cd, mask=m)
                new.append(cnts[t] + m.astype(jnp.int32))
            return tuple(new)

        cnts = lax.fori_loop(lo_s, hi_s, p2,
                             (jnp.zeros((LANES,), jnp.int32),) * NB)
        for t in range(NB):
            cnt_v[pl.ds((s * NB + t) * LANES, LANES)] = cnts[t]

    def box_body(bb, thrv):
        mx = jnp.max(cnt_v[pl.ds(bb * LANES, LANES)])
        u_b = u_sc[pl.ds(bb, LANES)][0]
        m_b = jnp.where(bb < NB, m0, m1)
        sent_ok = ((m_b == 0) | (u_b < FLOAT_MAX) | (m_b >= 33))

        def fast(_):
            r = []
            for q in range(4):
                cv = [can_v[pl.ds((bb * LCAP + q * 8 + k) * LANES, LANES)]
                      for k in range(8)]
                c0, c1, t = _select33v(cv)
                r += [c0, c1, jnp.full((LANES,), t, jnp.float32)]
            c0, c1, t = _squash33(r[0:6], inf16)
            d0, d1, t2 = _squash33(r[6:12], inf16)
            _, _, thr_b = _squash33(
                [c0, c1, jnp.full((LANES,), t, jnp.float32),
                 d0, d1, jnp.full((LANES,), t2, jnp.float32)], inf16)
            return thr_b

        def slow(_):
            gb = wid * 16 + bb
            bx = cx_v[pl.ds(gb, LANES)][0]
            by = cy_v[pl.ds(gb, LANES)][0]
            bz = cz_v[pl.ds(gb, LANES)][0]
            bl = bl_v[pl.ds(gb, LANES)][0]
            for j in range(8):
                buf_v[pl.ds(j * LANES, LANES)] = inf16

            def flush(thr, count):
                c0, c1, t = _select33v(
                    [buf_v[pl.ds(j * LANES, LANES)] for j in range(8)])
                buf_v[pl.ds(0, LANES)] = c0
                buf_v[pl.ds(LANES, LANES)] = c1
                buf_v[pl.ds(2 * LANES, LANES)] = jnp.full(
                    (LANES,), t, jnp.float32)
                for j in range(3, 8):
                    buf_v[pl.ds(j * LANES, LANES)] = inf16
                return t, jnp.int32(KEEP)

            def sbody(i, carry):
                thr, count = carry
                base = i * LANES
                dx = px_v[pl.ds(base, LANES)] - bx
                dy = py_v[pl.ds(base, LANES)] - by
                dz = pz_v[pl.ds(base, LANES)] - bz
                d = (dx * dx + dy * dy) + dz * dz
                cd = jnp.where(lev_v[pl.ds(base, LANES)] == bl, d, FLOAT_MAX)
                mask = cd < thr
                cnt2 = jnp.sum(mask.astype(jnp.int32))
                thr, count = lax.cond(count > CAP - LANES, flush,
                                      lambda t, c: (t, c), thr, count)
                plsc.store_compressed(buf_v.at[pl.ds(count, LANES)], cd,
                                      mask=mask)
                return thr, count + cnt2

            lax.fori_loop(0, NVREG, sbody,
                          (jnp.float32(jnp.inf), jnp.int32(0)))
            _, _, thr_b = _select33v(
                [buf_v[pl.ds(j * LANES, LANES)] for j in range(8)])
            return thr_b

        use_slow = (mx > LCAP) | jnp.logical_not(sent_ok)
        thr_b = lax.cond(use_slow, slow, fast, 0)
        adj = jnp.logical_not(use_slow) & (m_b > 0) & (u_b >= FLOAT_MAX)
        thr_b = jnp.where(adj, jnp.minimum(thr_b, FLOAT_MAX), thr_b)
        return jnp.where(lanes == bb, thr_b, thrv)

    thrv = lax.fori_loop(0, 16, box_body, inf16)
    stage_v[...] = thrv
    pltpu.sync_copy(stage_v, thr_h.at[pl.ds(wid * 16, 16)])


@functools.partial(
    pl.kernel,
    out_type=jax.ShapeDtypeStruct((N_PAD,), jnp.int32),
    mesh=_MESH,
    compiler_params=_PARAMS,
    scratch_types=[
        pltpu.VMEM((640,), jnp.float32),
        pltpu.VMEM((640,), jnp.float32),
        pltpu.VMEM((640,), jnp.float32),
        pltpu.VMEM((640,), jnp.int32),
        pltpu.VMEM((N_BOXES,), jnp.float32),
        pltpu.VMEM((N_BOXES,), jnp.float32),
        pltpu.VMEM((N_BOXES,), jnp.float32),
        pltpu.VMEM((N_BOXES,), jnp.int32),
        pltpu.VMEM((N_BOXES,), jnp.float32),
        pltpu.VMEM((640,), jnp.int32),
    ],
)
def _k2_assign(px_h, py_h, pz_h, lev_h, cx_h, cy_h, cz_h, bl_h, thr_in_h,
               out_h, px_v, py_v, pz_v, lev_v, cx_v, cy_v, cz_v, bl_v,
               thr_v, out_v):
    wid = lax.axis_index("s") * 2 + lax.axis_index("c")
    base = wid * 640
    pltpu.sync_copy(px_h.at[pl.ds(base, 640)], px_v)
    pltpu.sync_copy(py_h.at[pl.ds(base, 640)], py_v)
    pltpu.sync_copy(pz_h.at[pl.ds(base, 640)], pz_v)
    pltpu.sync_copy(lev_h.at[pl.ds(base, 640)], lev_v)
    pltpu.sync_copy(cx_h, cx_v)
    pltpu.sync_copy(cy_h, cy_v)
    pltpu.sync_copy(cz_h, cz_v)
    pltpu.sync_copy(bl_h, bl_v)
    pltpu.sync_copy(thr_in_h, thr_v)

    inf16 = jnp.full((LANES,), jnp.inf, jnp.float32)
    zero16 = jnp.zeros((LANES,), jnp.int32)
    VT = 4

    def vec_body(v, _):
        offs = [(v * VT + i) * LANES for i in range(VT)]
        pxs = [px_v[pl.ds(o, LANES)] for o in offs]
        pys = [py_v[pl.ds(o, LANES)] for o in offs]
        pzs = [pz_v[pl.ds(o, LANES)] for o in offs]
        levs = [lev_v[pl.ds(o, LANES)] for o in offs]

        def group_body(g, carry):
            mn, idx, mn2, idx2 = carry
            cxg = cx_v[pl.ds(g * LANES, LANES)]
            cyg = cy_v[pl.ds(g * LANES, LANES)]
            czg = cz_v[pl.ds(g * LANES, LANES)]
            blg = bl_v[pl.ds(g * LANES, LANES)]
            thg = thr_v[pl.ds(g * LANES, LANES)]
            mn, idx, mn2, idx2 = list(mn), list(idx), list(mn2), list(idx2)
            for j in range(LANES):
                b = g * LANES + j
                bx = cxg[j]
                by = cyg[j]
                bz = czg[j]
                bl = blg[j]
                th = thg[j]
                for i in range(VT):
                    dx = pxs[i] - bx
                    dy = pys[i] - by
                    dz = pzs[i] - bz
                    d = (dx * dx + dy * dy) + dz * dz
                    lt1 = d < mn[i]
                    mn[i] = jnp.where(lt1, d, mn[i])
                    idx[i] = jnp.where(lt1, b, idx[i])
                    cd = jnp.where(levs[i] == bl, d, FLOAT_MAX)
                    cd2 = jnp.where(cd < th, d, FLOAT_MAX)
                    lt2 = cd2 < mn2[i]
                    mn2[i] = jnp.where(lt2, cd2, mn2[i])
                    idx2[i] = jnp.where(lt2, b, idx2[i])
            return tuple(mn), tuple(idx), tuple(mn2), tuple(idx2)

        mn, idx, mn2, idx2 = lax.fori_loop(
            0, N_BOXES // LANES, group_body,
            ((inf16,) * VT, (zero16,) * VT, (inf16,) * VT, (zero16,) * VT))
        for i in range(VT):
            ok = (mn2[i] < FLOAT_MAX) & (idx2[i] == idx[i])
            out_v[pl.ds(offs[i], LANES)] = jnp.where(ok, idx2[i],
                                                     jnp.int32(-1))
        return _

    lax.fori_loop(0, 640 // (LANES * VT), vec_body, 0)
    pltpu.sync_copy(out_v, out_h.at[pl.ds(base, 640)])


def kernel(points, gt_boxes, levels, gt_labels):
    pad = N_PAD - N_PTS
    px = jnp.pad(points[:, 0], (0, pad))
    py = jnp.pad(points[:, 1], (0, pad))
    pz = jnp.pad(points[:, 2], (0, pad))
    lev = jnp.pad(levels, (0, pad), constant_values=-1)
    cx = gt_boxes[:, 0]
    cy = gt_boxes[:, 1]
    cz = gt_boxes[:, 2]
    bl = jnp.asarray([0], jnp.int32)[gt_labels]
    thr = _k1_thresholds(px, py, pz, lev, cx, cy, cz, bl)
    out = _k2_assign(px, py, pz, lev, cx, cy, cz, bl, thr)
    return out[:N_PTS]
